# baseline (speedup 1.0000x reference)
import functools

import jax
import jax.numpy as jnp
from jax import lax
from jax.experimental import pallas as pl
from jax.experimental.pallas import tpu as pltpu
from jax.experimental.pallas import tpu_sc as plsc

BSZ, BEAM, VOCAB = 64, 4, 100000
K = 8
L = 16

GELEM = 512
NFULL = 195
GP_BEAM = NFULL + 1
FULLV = NFULL * GELEM
NGROUP = BEAM * GP_BEAM
CAND = K * GELEM
RPS = 4

NW = 32
ROWS_PER_W = BSZ // NW

NEG = -3.0e38
IMAX = 2147483647


def _tc_body(lp_ref, bias_ref, cand_ref, gid_ref):
    gids = (jax.lax.broadcasted_iota(jnp.int32, (BEAM, GP_BEAM), 0) * GP_BEAM
            + jax.lax.broadcasted_iota(jnp.int32, (BEAM, GP_BEAM), 1))
    rows4 = jax.lax.broadcasted_iota(jnp.int32, (BEAM, 1), 0)
    io16 = jax.lax.broadcasted_iota(jnp.int32, (1, 1, L), 2)
    ms, tails, biases = [], [], []
    for r in range(RPS):
        x = lp_ref[r]
        b = bias_ref[r]
        m4 = jnp.max(x[:, :FULLV].reshape(BEAM, NFULL, 4, 128), axis=2)
        m195 = jnp.max(m4, axis=2)
        tail = jnp.max(x[:, FULLV:], axis=1, keepdims=True)
        ms.append(jnp.concatenate([m195, tail], axis=1) + b)
        tails.append(jnp.concatenate(
            [x[:, FULLV:],
             jnp.full((BEAM, GELEM - (VOCAB - FULLV)), NEG, jnp.float32)],
            axis=1))
        biases.append(b)
    gidvecs = [jnp.full((1, 1, L), IMAX, dtype=jnp.int32)] * RPS
    chunks = [[] for _ in range(RPS)]
    for k in range(K):
        for r in range(RPS):
            m = ms[r]
            gmaxv = jnp.max(m, axis=(0, 1), keepdims=True)
            gidv = jnp.min(jnp.where(m == gmaxv, gids, IMAX),
                           axis=(0, 1), keepdims=True)
            ms[r] = jnp.where(gids == gidv, NEG, m)
            beamv = gidv // GP_BEAM
            jv = gidv - beamv * GP_BEAM
            st = pl.multiple_of(
                jnp.minimum(jnp.min(jv), NFULL - 1) * GELEM, 128)
            win = lp_ref[r, :, pl.ds(st, GELEM)]
            chunk4 = jnp.where(jv >= NFULL, tails[r], win)
            b_k = jnp.max(jnp.where(rows4 == beamv, biases[r], NEG),
                          axis=(0, 1), keepdims=True)
            chunk = jnp.max(jnp.where(rows4 == beamv, chunk4, NEG),
                            axis=0, keepdims=True) + b_k
            chunks[r].append(chunk)
            gidvecs[r] = jnp.where(io16 == k, gidv.reshape(1, 1, 1),
                                   gidvecs[r])
    for r in range(RPS):
        cand_ref[r] = jnp.concatenate(chunks[r], axis=0)
        gid_ref[r] = gidvecs[r][0]


_tc_stage = pl.pallas_call(
    _tc_body,
    grid=(BSZ // RPS,),
    in_specs=[
        pl.BlockSpec((RPS, BEAM, VOCAB), lambda i: (i, 0, 0)),
        pl.BlockSpec((RPS, BEAM, 1), lambda i: (i, 0, 0)),
    ],
    out_specs=[
        pl.BlockSpec((RPS, K, GELEM), lambda i: (i, 0, 0)),
        pl.BlockSpec((RPS, 1, L), lambda i: (i, 0, 0)),
    ],
    out_shape=[
        jax.ShapeDtypeStruct((BSZ, K, GELEM), jnp.float32),
        jax.ShapeDtypeStruct((BSZ, 1, L), jnp.int32),
    ],
)


def _sc_body(cand_hbm, gid_hbm, vals_hbm, flats_hbm,
             cbuf, gbuf, ovst, ofst, sem0):
    wid = lax.axis_index("c") * 16 + lax.axis_index("s")
    iota = lax.iota(jnp.int32, L)
    negv = jnp.full((L,), NEG, dtype=jnp.float32)
    zerov = jnp.zeros((L,), dtype=jnp.int32)

    def row_body(r, _):
        row = wid * ROWS_PER_W + r
        c1 = pltpu.async_copy(cand_hbm.at[row], cbuf, sem0)
        pltpu.sync_copy(gid_hbm.at[row], gbuf)
        c1.wait()

        gv = gbuf[...]
        skeys, svals = plsc.sort_key_val(gv, iota)
        ov = jnp.zeros((L,), dtype=jnp.float32)
        of = zerov

        def lanemax(pk):
            def lm_step(v, lm):
                return jnp.maximum(lm, cbuf[pl.ds(pk * GELEM + v * L, L)])
            return lax.fori_loop(0, GELEM // L, lm_step, negv)

        S = negv
        for kk in range(K):
            pk = jnp.min(jnp.where(iota == kk, svals, IMAX))
            S = jnp.where(iota == kk, jnp.max(lanemax(pk)), S)

        for k in range(K):
            gmax = jnp.max(S)
            kstar = jnp.min(jnp.where(S == gmax, iota, IMAX))
            sid = jnp.min(jnp.where(iota == kstar, skeys, IMAX))
            pk = jnp.min(jnp.where(iota == kstar, svals, IMAX))
            beam = sid // GP_BEAM
            j = sid - beam * GP_BEAM
            base = beam * VOCAB + j * GELEM

            def minix_step(v, mi, pk=pk, base=base, gmax=gmax):
                x = cbuf[pl.ds(pk * GELEM + v * L, L)]
                ixv = base + v * L + iota
                return jnp.minimum(mi, jnp.where(x == gmax, ixv, IMAX))

            minIX = lax.fori_loop(0, GELEM // L, minix_step,
                                  jnp.full((L,), IMAX, dtype=jnp.int32))
            istar = jnp.min(minIX)
            pstar = pk * GELEM + (istar - base)
            plsc.store_scatter(cbuf, [jnp.broadcast_to(pstar, (L,))], negv,
                               mask=iota == 0)
            S = jnp.where(iota == kstar, jnp.max(lanemax(pk)), S)
            ov = jnp.where(iota == k, gmax, ov)
            of = jnp.where(iota == k, istar, of)

        ovst[...] = ov
        ofst[...] = of
        pltpu.sync_copy(ovst, vals_hbm.at[row])
        pltpu.sync_copy(ofst, flats_hbm.at[row])
        return 0

    lax.fori_loop(0, ROWS_PER_W, row_body, 0)


_sc_stage = pl.kernel(
    _sc_body,
    out_type=[
        jax.ShapeDtypeStruct((BSZ, L), jnp.float32),
        jax.ShapeDtypeStruct((BSZ, L), jnp.int32),
    ],
    mesh=plsc.VectorSubcoreMesh(core_axis_name="c", subcore_axis_name="s",
                                num_cores=2, num_subcores=16),
    compiler_params=pltpu.CompilerParams(use_tc_tiling_on_sc=False,
                                         needs_layout_passes=False),
    scratch_types=[
        pltpu.VMEM((CAND,), jnp.float32),
        pltpu.VMEM((L,), jnp.int32),
        pltpu.VMEM((L,), jnp.float32),
        pltpu.VMEM((L,), jnp.int32),
        pltpu.SemaphoreType.DMA,
    ],
)


def kernel(step, lprobs, scores):
    bsz, beam, vocab = lprobs.shape
    bias = jnp.take(scores, step - 1, axis=2)
    cand, gids = _tc_stage(lprobs, bias[:, :, None])
    vals, flats = _sc_stage(cand.reshape(bsz, CAND),
                            gids.reshape(bsz, L))
    vals = vals[:, :K]
    flats = flats[:, :K]
    return (vals, flats % vocab, flats // vocab)


# speedup vs baseline: 1.0807x; 1.0807x over previous
import functools

import jax
import jax.numpy as jnp
from jax import lax
from jax.experimental import pallas as pl
from jax.experimental.pallas import tpu as pltpu
from jax.experimental.pallas import tpu_sc as plsc

BSZ, BEAM, VOCAB = 64, 4, 100000
K = 8
L = 16

GELEM = 512
GP_BEAM = 196
VPAD = GP_BEAM * GELEM
NGROUP = BEAM * GP_BEAM
CAND = K * GELEM

NW = 32
ROWS_PER_W = BSZ // NW

NEG = -3.0e38
IMAX = 2147483647


def _tc_body(lp_ref, bias_ref, cand_ref, gid_ref, y2_ref):
    x = lp_ref[0]
    b = bias_ref[0]
    y2_ref[:, :VOCAB] = x + b
    y2_ref[:, VOCAB:] = jnp.full((BEAM, VPAD - VOCAB), NEG, dtype=jnp.float32)
    y2 = y2_ref[...]
    m = jnp.max(y2.reshape(BEAM, GP_BEAM, GELEM), axis=2)
    gids = (jax.lax.broadcasted_iota(jnp.int32, (BEAM, GP_BEAM), 0) * GP_BEAM
            + jax.lax.broadcasted_iota(jnp.int32, (BEAM, GP_BEAM), 1))
    rows4 = jax.lax.broadcasted_iota(jnp.int32, (BEAM, 1), 0)
    io16 = jax.lax.broadcasted_iota(jnp.int32, (1, 1, L), 2)
    gidvec = jnp.full((1, 1, L), IMAX, dtype=jnp.int32)
    for k in range(K):
        gmax = jnp.max(m)
        gid = jnp.min(jnp.where(m == gmax, gids, IMAX))
        m = jnp.where(gids == gid, NEG, m)
        beam = gid // GP_BEAM
        j = gid % GP_BEAM
        chunk4 = y2_ref[:, pl.ds(j * GELEM, GELEM)]
        chunk = jnp.max(jnp.where(rows4 == beam, chunk4, NEG), axis=0)
        cand_ref[0, k] = chunk
        gidvec = jnp.where(io16 == k, gid, gidvec)
    gid_ref[...] = gidvec


_tc_stage = pl.pallas_call(
    _tc_body,
    grid=(BSZ,),
    in_specs=[
        pl.BlockSpec((1, BEAM, VOCAB), lambda i: (i, 0, 0)),
        pl.BlockSpec((1, BEAM, 1), lambda i: (i, 0, 0)),
    ],
    out_specs=[
        pl.BlockSpec((1, K, GELEM), lambda i: (i, 0, 0)),
        pl.BlockSpec((1, 1, L), lambda i: (i, 0, 0)),
    ],
    out_shape=[
        jax.ShapeDtypeStruct((BSZ, K, GELEM), jnp.float32),
        jax.ShapeDtypeStruct((BSZ, 1, L), jnp.int32),
    ],
    scratch_shapes=[pltpu.VMEM((BEAM, VPAD), jnp.float32)],
)


def _sc_body(cand_hbm, gid_hbm, vals_hbm, flats_hbm,
             cbuf, gbuf, ovst, ofst, sem0):
    wid = lax.axis_index("c") * 16 + lax.axis_index("s")
    iota = lax.iota(jnp.int32, L)
    negv = jnp.full((L,), NEG, dtype=jnp.float32)
    zerov = jnp.zeros((L,), dtype=jnp.int32)

    def row_body(r, _):
        row = wid * ROWS_PER_W + r
        c1 = pltpu.async_copy(cand_hbm.at[row], cbuf, sem0)
        pltpu.sync_copy(gid_hbm.at[row], gbuf)
        c1.wait()

        gv = gbuf[...]
        skeys, svals = plsc.sort_key_val(gv, iota)
        ov = jnp.zeros((L,), dtype=jnp.float32)
        of = zerov

        def lanemax(pk):
            def lm_step(v, lm):
                return jnp.maximum(lm, cbuf[pl.ds(pk * GELEM + v * L, L)])
            return lax.fori_loop(0, GELEM // L, lm_step, negv)

        S = negv
        for kk in range(K):
            pk = jnp.min(jnp.where(iota == kk, svals, IMAX))
            S = jnp.where(iota == kk, jnp.max(lanemax(pk)), S)

        for k in range(K):
            gmax = jnp.max(S)
            kstar = jnp.min(jnp.where(S == gmax, iota, IMAX))
            sid = jnp.min(jnp.where(iota == kstar, skeys, IMAX))
            pk = jnp.min(jnp.where(iota == kstar, svals, IMAX))
            beam = sid // GP_BEAM
            j = sid - beam * GP_BEAM
            base = beam * VOCAB + j * GELEM

            def minix_step(v, mi, pk=pk, base=base, gmax=gmax):
                x = cbuf[pl.ds(pk * GELEM + v * L, L)]
                ixv = base + v * L + iota
                return jnp.minimum(mi, jnp.where(x == gmax, ixv, IMAX))

            minIX = lax.fori_loop(0, GELEM // L, minix_step,
                                  jnp.full((L,), IMAX, dtype=jnp.int32))
            istar = jnp.min(minIX)
            pstar = pk * GELEM + (istar - base)
            plsc.store_scatter(cbuf, [jnp.broadcast_to(pstar, (L,))], negv,
                               mask=iota == 0)
            S = jnp.where(iota == kstar, jnp.max(lanemax(pk)), S)
            ov = jnp.where(iota == k, gmax, ov)
            of = jnp.where(iota == k, istar, of)

        ovst[...] = ov
        ofst[...] = of
        pltpu.sync_copy(ovst, vals_hbm.at[row])
        pltpu.sync_copy(ofst, flats_hbm.at[row])
        return 0

    lax.fori_loop(0, ROWS_PER_W, row_body, 0)


_sc_stage = pl.kernel(
    _sc_body,
    out_type=[
        jax.ShapeDtypeStruct((BSZ, L), jnp.float32),
        jax.ShapeDtypeStruct((BSZ, L), jnp.int32),
    ],
    mesh=plsc.VectorSubcoreMesh(core_axis_name="c", subcore_axis_name="s",
                                num_cores=2, num_subcores=16),
    compiler_params=pltpu.CompilerParams(use_tc_tiling_on_sc=False,
                                         needs_layout_passes=False),
    scratch_types=[
        pltpu.VMEM((CAND,), jnp.float32),
        pltpu.VMEM((L,), jnp.int32),
        pltpu.VMEM((L,), jnp.float32),
        pltpu.VMEM((L,), jnp.int32),
        pltpu.SemaphoreType.DMA,
    ],
)


def kernel(step, lprobs, scores):
    bsz, beam, vocab = lprobs.shape
    bias = jnp.take(scores, step - 1, axis=2)
    cand, gids = _tc_stage(lprobs, bias[:, :, None])
    vals, flats = _sc_stage(cand.reshape(bsz, CAND),
                            gids.reshape(bsz, L))
    vals = vals[:, :K]
    flats = flats[:, :K]
    return (vals, flats % vocab, flats // vocab)


# speedup vs baseline: 1.0814x; 1.0007x over previous
import jax
import jax.numpy as jnp
from jax import lax
from jax.experimental import pallas as pl
from jax.experimental.pallas import tpu as pltpu
from jax.experimental.pallas import tpu_sc as plsc

BSZ, BEAM, VOCAB = 64, 4, 100000
K = 8
L = 16

GELEM = 512
GP_BEAM = 196
VPAD = GP_BEAM * GELEM
NGROUP = BEAM * GP_BEAM
CAND = K * GELEM

NW = 32
ROWS_PER_W = BSZ // NW

NEG = -3.0e38
IMAX = 2147483647


def _tc_body(lp_ref, bias_ref, cand_ref, gid_ref, y2_ref):
    x = lp_ref[0]
    b = bias_ref[0]
    y2_ref[:, :VOCAB] = x + b
    y2_ref[:, VOCAB:] = jnp.full((BEAM, VPAD - VOCAB), NEG, dtype=jnp.float32)
    y2 = y2_ref[...]
    m = jnp.max(y2.reshape(BEAM, GP_BEAM, GELEM), axis=2)
    gids = (jax.lax.broadcasted_iota(jnp.int32, (BEAM, GP_BEAM), 0) * GP_BEAM
            + jax.lax.broadcasted_iota(jnp.int32, (BEAM, GP_BEAM), 1))
    rows4 = jax.lax.broadcasted_iota(jnp.int32, (BEAM, 1), 0)
    io16 = jax.lax.broadcasted_iota(jnp.int32, (1, 1, L), 2)
    gidvec = jnp.full((1, 1, L), IMAX, dtype=jnp.int32)
    for k in range(K):
        gmax = jnp.max(m)
        gid = jnp.min(jnp.where(m == gmax, gids, IMAX))
        m = jnp.where(gids == gid, NEG, m)
        beam = gid // GP_BEAM
        j = gid % GP_BEAM
        chunk4 = y2_ref[:, pl.ds(j * GELEM, GELEM)]
        chunk = jnp.max(jnp.where(rows4 == beam, chunk4, NEG), axis=0)
        cand_ref[0, k] = chunk
        gidvec = jnp.where(io16 == k, gid, gidvec)
    gid_ref[...] = gidvec


_tc_stage = pl.pallas_call(
    _tc_body,
    grid=(BSZ,),
    in_specs=[
        pl.BlockSpec((1, BEAM, VOCAB), lambda i: (i, 0, 0)),
        pl.BlockSpec((1, BEAM, 1), lambda i: (i, 0, 0)),
    ],
    out_specs=[
        pl.BlockSpec((1, K, GELEM), lambda i: (i, 0, 0)),
        pl.BlockSpec((1, 1, L), lambda i: (i, 0, 0)),
    ],
    out_shape=[
        jax.ShapeDtypeStruct((BSZ, K, GELEM), jnp.float32),
        jax.ShapeDtypeStruct((BSZ, 1, L), jnp.int32),
    ],
    scratch_shapes=[pltpu.VMEM((BEAM, VPAD), jnp.float32)],
)


def _sc_body(cand_hbm, gid_hbm, vals_hbm, flats_hbm,
             cbuf, gbuf, ovst, ofst, sem0):
    wid = lax.axis_index("c") * 16 + lax.axis_index("s")
    iota = lax.iota(jnp.int32, L)
    negv = jnp.full((L,), NEG, dtype=jnp.float32)
    zerov = jnp.zeros((L,), dtype=jnp.int32)

    def row_body(r, _):
        row = wid * ROWS_PER_W + r
        c1 = pltpu.async_copy(cand_hbm.at[row], cbuf, sem0)
        pltpu.sync_copy(gid_hbm.at[row], gbuf)
        c1.wait()

        gv = gbuf[...]
        skeys, svals = plsc.sort_key_val(gv, iota)
        ov = jnp.zeros((L,), dtype=jnp.float32)
        of = zerov

        def lanemax(pk):
            def lm_step(v, lm):
                return jnp.maximum(lm, cbuf[pl.ds(pk * GELEM + v * L, L)])
            return lax.fori_loop(0, GELEM // L, lm_step, negv)

        S = negv
        for kk in range(K):
            pk = jnp.min(jnp.where(iota == kk, svals, IMAX))
            S = jnp.where(iota == kk, jnp.max(lanemax(pk)), S)

        for k in range(K):
            gmax = jnp.max(S)
            kstar = jnp.min(jnp.where(S == gmax, iota, IMAX))
            sid = jnp.min(jnp.where(iota == kstar, skeys, IMAX))
            pk = jnp.min(jnp.where(iota == kstar, svals, IMAX))
            beam = sid // GP_BEAM
            j = sid - beam * GP_BEAM
            base = beam * VOCAB + j * GELEM

            def minix_step(v, mi, pk=pk, base=base, gmax=gmax):
                x = cbuf[pl.ds(pk * GELEM + v * L, L)]
                ixv = base + v * L + iota
                return jnp.minimum(mi, jnp.where(x == gmax, ixv, IMAX))

            minIX = lax.fori_loop(0, GELEM // L, minix_step,
                                  jnp.full((L,), IMAX, dtype=jnp.int32))
            istar = jnp.min(minIX)
            pstar = pk * GELEM + (istar - base)
            plsc.store_scatter(cbuf, [jnp.broadcast_to(pstar, (L,))], negv,
                               mask=iota == 0)
            S = jnp.where(iota == kstar, jnp.max(lanemax(pk)), S)
            ov = jnp.where(iota == k, gmax, ov)
            of = jnp.where(iota == k, istar, of)

        ovst[...] = ov
        ofst[...] = of
        pltpu.sync_copy(ovst, vals_hbm.at[row])
        pltpu.sync_copy(ofst, flats_hbm.at[row])
        return 0

    lax.fori_loop(0, ROWS_PER_W, row_body, 0)


_sc_stage = pl.kernel(
    _sc_body,
    out_type=[
        jax.ShapeDtypeStruct((BSZ, L), jnp.float32),
        jax.ShapeDtypeStruct((BSZ, L), jnp.int32),
    ],
    mesh=plsc.VectorSubcoreMesh(core_axis_name="c", subcore_axis_name="s",
                                num_cores=2, num_subcores=16),
    compiler_params=pltpu.CompilerParams(use_tc_tiling_on_sc=False,
                                         needs_layout_passes=False),
    scratch_types=[
        pltpu.VMEM((CAND,), jnp.float32),
        pltpu.VMEM((L,), jnp.int32),
        pltpu.VMEM((L,), jnp.float32),
        pltpu.VMEM((L,), jnp.int32),
        pltpu.SemaphoreType.DMA,
    ],
)


def kernel(step, lprobs, scores):
    bsz, beam, vocab = lprobs.shape
    bias = jnp.take(scores, step - 1, axis=2)
    cand, gids = _tc_stage(lprobs, bias[:, :, None])
    vals, flats = _sc_stage(cand.reshape(bsz, CAND),
                            gids.reshape(bsz, L))
    vals = vals[:, :K]
    flats = flats[:, :K]
    return (vals, flats % vocab, flats // vocab)
